# baseline (speedup 1.0000x reference)
import functools
import math

import jax
import jax.numpy as jnp
from jax import lax
from jax.experimental import pallas as pl
from jax.experimental.pallas import tpu as pltpu

_GC1 = math.sqrt(2.0 / math.pi)
_GC2 = _GC1 * 0.044715


def _ln(x, g, b, eps=1e-5):
    mu = jnp.mean(x, axis=-1, keepdims=True)
    xc = x - mu
    v = jnp.mean(xc * xc, axis=-1, keepdims=True)
    return xc * lax.rsqrt(v + eps) * g + b


def _gelu(x):
    t = jnp.tanh(x * (_GC1 + _GC2 * (x * x)))
    return (0.5 * x) * (1.0 + t)


def _ceil_to(x, m):
    return (x + m - 1) // m * m


def _entity_body(x_ref, wcat_ref, bcat_ref, wp2_ref, bp2_ref, opc_ref,
                 wra_ref, wrb_ref, br1_ref,
                 wd1_ref, bd1_ref, g1_ref, e1_ref,
                 wd2_ref, bd2_ref, g2_ref, e2_ref,
                 st_ref, ph_ref, ac_ref, bc_ref, va_ref, vb_ref, dec_ref,
                 *, S):
    x = x_ref[...]
    t = jnp.tanh(jnp.dot(x, wcat_ref[...],
                         preferred_element_type=jnp.float32) + bcat_ref[...])
    re = t[:, :S]
    im = t[:, S:2 * S]
    ph = math.pi * jnp.tanh(
        jnp.dot(t[:, 2 * S:], wp2_ref[...],
                preferred_element_type=jnp.float32) + bp2_ref[...])
    cp = jnp.cos(ph)
    sp = jnp.sin(ph)
    rw = re * cp - im * sp
    iw = re * sp + im * cp
    inv = lax.rsqrt(jnp.sum(rw * rw + iw * iw, axis=-1, keepdims=True) + 1e-12)
    st = jnp.dot(jnp.concatenate([rw * inv, iw * inv], axis=-1), opc_ref[...],
                 preferred_element_type=jnp.float32)
    st_ref[...] = st
    ph_ref[...] = ph

    real = st[:, :S]
    a = jnp.dot(real, wra_ref[...],
                preferred_element_type=jnp.float32) + br1_ref[...]
    b = jnp.dot(real, wrb_ref[...], preferred_element_type=jnp.float32)
    ac = a - jnp.mean(a, axis=-1, keepdims=True)
    bc = b - jnp.mean(b, axis=-1, keepdims=True)
    ac_ref[...] = ac
    bc_ref[...] = bc
    va_ref[...] = jnp.mean(ac * ac, axis=-1, keepdims=True)
    vb_ref[...] = jnp.mean(bc * bc, axis=-1, keepdims=True)

    hd = jnp.dot(real, wd1_ref[...],
                 preferred_element_type=jnp.float32) + bd1_ref[...]
    hd = _gelu(_ln(hd, g1_ref[...], e1_ref[...]))
    y = jnp.dot(hd.astype(jnp.bfloat16), wd2_ref[...],
                preferred_element_type=jnp.float32) + bd2_ref[...]
    dec_ref[...] = _ln(y, g2_ref[...], e2_ref[...])


def _rel_body(ac_ref, bc_ref, va_ref, vb_ref, g_ref, be_ref, w2_ref,
              b2t_ref,
              out_ref, sc_ref, cf_ref, *, TI, TJ, H, Q):
    bc = bc_ref[...]
    gram = lax.dot_general(bc, ac_ref[...],
                           (((1,), (1,)), ((), ())),
                           preferred_element_type=jnp.float32)
    rstd = lax.rsqrt(gram * (2.0 / H) + vb_ref[...] + va_ref[0] + 1e-5)

    g = g_ref[...]
    be = be_ref[...]
    w2 = w2_ref[...]
    b2t = b2t_ref[...]
    j0 = pl.program_id(1) * TJ
    jota = lax.broadcasted_iota(jnp.int32, (1, TJ), 1)
    for ii in range(TI):
        xc = ac_ref[ii:ii + 1, :] + bc
        xn = (xc * rstd[:, ii:ii + 1]) * g + be
        u = _gelu(xn)
        ot = lax.dot_general(w2, u, (((0,), (1,)), ((), ())),
                             preferred_element_type=jnp.float32) + b2t
        out_ref[ii] = ot
        sc_ref[ii:ii + 1, :] = jnp.sum(ot, axis=0, keepdims=True)
        nrm = jnp.sqrt(jnp.sum(ot * ot, axis=0, keepdims=True)) * \
            (1.0 / math.sqrt(Q))
        cf = jnp.minimum(nrm, 1.0)
        gi = pl.program_id(0) * TI + ii
        cf_ref[ii:ii + 1, :] = jnp.where(jota == gi - j0, 0.0, cf)


def kernel(entity_emb, w_real, b_real, w_imag, b_imag, w_phase1, b_phase1,
           w_phase2, b_phase2, op_real, op_imag, op_blocks,
           rel_w1, rel_b1, rel_ln_g, rel_ln_b, rel_w2, rel_b2,
           dec_w1, dec_b1, dec_ln1_g, dec_ln1_b, dec_w2, dec_b2,
           dec_ln2_g, dec_ln2_b, normalization):
    x = jnp.asarray(entity_emb, jnp.float32)
    N, E = x.shape
    S = w_phase2.shape[0]
    H = rel_w1.shape[1]
    Q = rel_w2.shape[1]

    w_cat = jnp.concatenate([w_real, w_imag, w_phase1], axis=1)
    b_cat = jnp.concatenate([b_real, b_imag, b_phase1], axis=1)
    opc = op_blocks[0]
    for s in range(1, op_blocks.shape[0]):
        opc = jnp.dot(opc, op_blocks[s], preferred_element_type=jnp.float32)
    wra = rel_w1[:S, :]
    wrb = rel_w1[S:, :]
    wd2_bf = dec_w2.astype(jnp.bfloat16)

    TM = 128 if N % 128 == 0 else 8
    Np = _ceil_to(max(N, 8), TM)
    xp = x if Np == N else jnp.zeros((Np, E), jnp.float32).at[:N].set(x)

    full = lambda i: (0, 0)
    rowb = lambda i: (i, 0)
    st, ph, ac, bc, va, vb, dec = pl.pallas_call(
        functools.partial(_entity_body, S=S),
        out_shape=(jax.ShapeDtypeStruct((Np, 2 * S), jnp.float32),
                   jax.ShapeDtypeStruct((Np, S), jnp.float32),
                   jax.ShapeDtypeStruct((Np, H), jnp.float32),
                   jax.ShapeDtypeStruct((Np, H), jnp.float32),
                   jax.ShapeDtypeStruct((Np, 1), jnp.float32),
                   jax.ShapeDtypeStruct((Np, 1), jnp.float32),
                   jax.ShapeDtypeStruct((Np, E), jnp.float32)),
        grid=(Np // TM,),
        in_specs=[pl.BlockSpec((TM, E), rowb),
                  pl.BlockSpec((E, 3 * S), full),
                  pl.BlockSpec((1, 3 * S), full),
                  pl.BlockSpec((S, S), full),
                  pl.BlockSpec((1, S), full),
                  pl.BlockSpec((2 * S, 2 * S), full),
                  pl.BlockSpec((S, H), full),
                  pl.BlockSpec((S, H), full),
                  pl.BlockSpec((1, H), full),
                  pl.BlockSpec((S, H), full),
                  pl.BlockSpec((1, H), full),
                  pl.BlockSpec((1, H), full),
                  pl.BlockSpec((1, H), full),
                  pl.BlockSpec((H, E), full),
                  pl.BlockSpec((1, E), full),
                  pl.BlockSpec((1, E), full),
                  pl.BlockSpec((1, E), full)],
        out_specs=[pl.BlockSpec((TM, 2 * S), rowb),
                   pl.BlockSpec((TM, S), rowb),
                   pl.BlockSpec((TM, H), rowb),
                   pl.BlockSpec((TM, H), rowb),
                   pl.BlockSpec((TM, 1), rowb),
                   pl.BlockSpec((TM, 1), rowb),
                   pl.BlockSpec((TM, E), rowb)],
        compiler_params=pltpu.CompilerParams(dimension_semantics=("parallel",)),
        cost_estimate=pl.CostEstimate(
            flops=Np * 2 * (E * 3 * S + S * S + 4 * S * S + 3 * S * H + H * E),
            transcendentals=Np * (6 * S + H),
            bytes_accessed=4 * Np * (2 * E + 3 * S + 2 * H) + 4 * E * 3 * S
            + 2 * H * E + 4 * 3 * S * H),
    )(xp, w_cat, b_cat, w_phase2, b_phase2, opc, wra, wrb, rel_b1,
      dec_w1, dec_b1, dec_ln1_g, dec_ln1_b, wd2_bf, dec_b2,
      dec_ln2_g, dec_ln2_b)

    TI = 32 if N % 32 == 0 else 8
    TJ = 512 if N % 512 == 0 else (256 if N % 256 == 0 else
                                   (128 if N % 128 == 0 else N))
    Npr = _ceil_to(max(N, TI), TI)
    Npc = _ceil_to(max(N, TJ), TJ)
    Ap = ac[:N] if Npr == N else (
        jnp.zeros((Npr, H), jnp.float32).at[:N].set(ac[:N]))
    Bp = bc[:N] if Npc == N else (
        jnp.zeros((Npc, H), jnp.float32).at[:N].set(bc[:N]))
    vap = va[:N] if Npr == N else (
        jnp.zeros((Npr, 1), jnp.float32).at[:N].set(va[:N]))
    vbp = vb[:N] if Npc == N else (
        jnp.zeros((Npc, 1), jnp.float32).at[:N].set(vb[:N]))
    va3 = vap.reshape(Npr // TI, 1, TI)

    relt, sc, cf = pl.pallas_call(
        functools.partial(_rel_body, TI=TI, TJ=TJ, H=H, Q=Q),
        out_shape=(jax.ShapeDtypeStruct((Npr, Q, Npc), jnp.float32),
                   jax.ShapeDtypeStruct((Npr, Npc), jnp.float32),
                   jax.ShapeDtypeStruct((Npr, Npc), jnp.float32)),
        grid=(Npr // TI, Npc // TJ),
        in_specs=[pl.BlockSpec((TI, H), lambda i, j: (i, 0)),
                  pl.BlockSpec((TJ, H), lambda i, j: (j, 0)),
                  pl.BlockSpec((1, 1, TI), lambda i, j: (i, 0, 0)),
                  pl.BlockSpec((TJ, 1), lambda i, j: (j, 0)),
                  pl.BlockSpec((1, H), lambda i, j: (0, 0)),
                  pl.BlockSpec((1, H), lambda i, j: (0, 0)),
                  pl.BlockSpec((H, Q), lambda i, j: (0, 0)),
                  pl.BlockSpec((Q, 1), lambda i, j: (0, 0))],
        out_specs=[pl.BlockSpec((TI, Q, TJ), lambda i, j: (i, 0, j)),
                   pl.BlockSpec((TI, TJ), lambda i, j: (i, j)),
                   pl.BlockSpec((TI, TJ), lambda i, j: (i, j))],
        compiler_params=pltpu.CompilerParams(
            dimension_semantics=("parallel", "parallel")),
        cost_estimate=pl.CostEstimate(
            flops=Npr * Npc * (2 * H * Q + 14 * H + 4 * Q + 2 * H),
            transcendentals=Npr * Npc * H,
            bytes_accessed=4 * (Npr * H + Npc * H + Npr * Npc * (Q + 2))),
    )(Ap, Bp, va3, vbp, rel_ln_g, rel_ln_b, rel_w2,
      jnp.transpose(rel_b2))

    rel = jnp.transpose(relt, (0, 2, 1))
    if Npr != N or Npc != N:
        rel = rel[:N, :N]
        sc = sc[:N, :N]
        cf = cf[:N, :N]
    enhanced = dec if Np == N else dec[:N]
    meta = {"relation_states": rel,
            "relation_scores": sc,
            "confidence": cf,
            "quantum_states": {"real": st[:N, :S], "imag": st[:N, S:],
                               "phases": ph if Np == N else ph[:N]}}
    return enhanced, meta


# speedup vs baseline: 1.0307x; 1.0307x over previous
import functools
import math

import jax
import jax.numpy as jnp
from jax import lax
from jax.experimental import pallas as pl
from jax.experimental.pallas import tpu as pltpu

_GC1 = math.sqrt(2.0 / math.pi)
_GC2 = _GC1 * 0.044715


def _ln(x, g, b, eps=1e-5):
    mu = jnp.mean(x, axis=-1, keepdims=True)
    xc = x - mu
    v = jnp.mean(xc * xc, axis=-1, keepdims=True)
    return xc * lax.rsqrt(v + eps) * g + b


def _gelu(x):
    t = jnp.tanh(x * (_GC1 + _GC2 * (x * x)))
    return (0.5 * x) * (1.0 + t)


def _ceil_to(x, m):
    return (x + m - 1) // m * m


def _entity_body(x_ref, wcat_ref, bcat_ref, wp2_ref, bp2_ref, opc_ref,
                 wra_ref, wrb_ref, br1_ref,
                 wd1_ref, bd1_ref, g1_ref, e1_ref,
                 wd2_ref, bd2_ref, g2_ref, e2_ref,
                 st_ref, ph_ref, ac_ref, bc_ref, va_ref, vb_ref, dec_ref,
                 *, S):
    x = x_ref[...]
    t = jnp.tanh(jnp.dot(x.astype(jnp.bfloat16), wcat_ref[...],
                         preferred_element_type=jnp.float32) + bcat_ref[...])
    re = t[:, :S]
    im = t[:, S:2 * S]
    ph = math.pi * jnp.tanh(
        jnp.dot(t[:, 2 * S:], wp2_ref[...],
                preferred_element_type=jnp.float32) + bp2_ref[...])
    cp = jnp.cos(ph)
    sp = jnp.sin(ph)
    rw = re * cp - im * sp
    iw = re * sp + im * cp
    inv = lax.rsqrt(jnp.sum(rw * rw + iw * iw, axis=-1, keepdims=True) + 1e-12)
    st = jnp.dot(jnp.concatenate([rw * inv, iw * inv], axis=-1), opc_ref[...],
                 preferred_element_type=jnp.float32)
    st_ref[...] = st
    ph_ref[...] = ph

    real = st[:, :S]
    a = jnp.dot(real, wra_ref[...],
                preferred_element_type=jnp.float32) + br1_ref[...]
    b = jnp.dot(real, wrb_ref[...], preferred_element_type=jnp.float32)
    ac = a - jnp.mean(a, axis=-1, keepdims=True)
    bc = b - jnp.mean(b, axis=-1, keepdims=True)
    ac_ref[...] = ac
    bc_ref[...] = bc
    va_ref[...] = jnp.mean(ac * ac, axis=-1, keepdims=True)
    vb_ref[...] = jnp.mean(bc * bc, axis=-1, keepdims=True)

    hd = jnp.dot(real, wd1_ref[...],
                 preferred_element_type=jnp.float32) + bd1_ref[...]
    hd = _gelu(_ln(hd, g1_ref[...], e1_ref[...]))
    y = jnp.dot(hd.astype(jnp.bfloat16), wd2_ref[...],
                preferred_element_type=jnp.float32) + bd2_ref[...]
    dec_ref[...] = _ln(y, g2_ref[...], e2_ref[...])


def _rel_body(ac_ref, bc_ref, va_ref, vb_ref, g_ref, be_ref, w2_ref,
              b2t_ref,
              out_ref, sc_ref, cf_ref, *, TI, TJ, H, Q):
    bc = bc_ref[...]
    gram = lax.dot_general(bc, ac_ref[...],
                           (((1,), (1,)), ((), ())),
                           preferred_element_type=jnp.float32)
    rstd = lax.rsqrt(gram * (2.0 / H) + vb_ref[...] + va_ref[0] + 1e-5)

    g = g_ref[...]
    be = be_ref[...]
    w2 = w2_ref[...]
    b2t = b2t_ref[...]
    j0 = pl.program_id(1) * TJ
    jota = lax.broadcasted_iota(jnp.int32, (1, TJ), 1)
    for ii in range(TI):
        xc = ac_ref[ii:ii + 1, :] + bc
        xn = (xc * rstd[:, ii:ii + 1]) * g + be
        u = _gelu(xn)
        ot = lax.dot_general(w2, u, (((0,), (1,)), ((), ())),
                             preferred_element_type=jnp.float32) + b2t
        out_ref[ii] = ot
        sc_ref[ii:ii + 1, :] = jnp.sum(ot, axis=0, keepdims=True)
        nrm = jnp.sqrt(jnp.sum(ot * ot, axis=0, keepdims=True)) * \
            (1.0 / math.sqrt(Q))
        cf = jnp.minimum(nrm, 1.0)
        gi = pl.program_id(0) * TI + ii
        cf_ref[ii:ii + 1, :] = jnp.where(jota == gi - j0, 0.0, cf)


def kernel(entity_emb, w_real, b_real, w_imag, b_imag, w_phase1, b_phase1,
           w_phase2, b_phase2, op_real, op_imag, op_blocks,
           rel_w1, rel_b1, rel_ln_g, rel_ln_b, rel_w2, rel_b2,
           dec_w1, dec_b1, dec_ln1_g, dec_ln1_b, dec_w2, dec_b2,
           dec_ln2_g, dec_ln2_b, normalization):
    x = jnp.asarray(entity_emb, jnp.float32)
    N, E = x.shape
    S = w_phase2.shape[0]
    H = rel_w1.shape[1]
    Q = rel_w2.shape[1]

    w_cat = jnp.concatenate([w_real, w_imag, w_phase1], axis=1)
    b_cat = jnp.concatenate([b_real, b_imag, b_phase1], axis=1)
    opc = op_blocks[0]
    for s in range(1, op_blocks.shape[0]):
        opc = jnp.dot(opc, op_blocks[s], preferred_element_type=jnp.float32)
    wra = rel_w1[:S, :]
    wrb = rel_w1[S:, :]
    wd2_bf = dec_w2.astype(jnp.bfloat16)

    TM = 128 if N % 128 == 0 else 8
    Np = _ceil_to(max(N, 8), TM)
    xp = x if Np == N else jnp.zeros((Np, E), jnp.float32).at[:N].set(x)

    full = lambda i: (0, 0)
    rowb = lambda i: (i, 0)
    st, ph, ac, bc, va, vb, dec = pl.pallas_call(
        functools.partial(_entity_body, S=S),
        out_shape=(jax.ShapeDtypeStruct((Np, 2 * S), jnp.float32),
                   jax.ShapeDtypeStruct((Np, S), jnp.float32),
                   jax.ShapeDtypeStruct((Np, H), jnp.float32),
                   jax.ShapeDtypeStruct((Np, H), jnp.float32),
                   jax.ShapeDtypeStruct((Np, 1), jnp.float32),
                   jax.ShapeDtypeStruct((Np, 1), jnp.float32),
                   jax.ShapeDtypeStruct((Np, E), jnp.float32)),
        grid=(Np // TM,),
        in_specs=[pl.BlockSpec((TM, E), rowb),
                  pl.BlockSpec((E, 3 * S), full),
                  pl.BlockSpec((1, 3 * S), full),
                  pl.BlockSpec((S, S), full),
                  pl.BlockSpec((1, S), full),
                  pl.BlockSpec((2 * S, 2 * S), full),
                  pl.BlockSpec((S, H), full),
                  pl.BlockSpec((S, H), full),
                  pl.BlockSpec((1, H), full),
                  pl.BlockSpec((S, H), full),
                  pl.BlockSpec((1, H), full),
                  pl.BlockSpec((1, H), full),
                  pl.BlockSpec((1, H), full),
                  pl.BlockSpec((H, E), full),
                  pl.BlockSpec((1, E), full),
                  pl.BlockSpec((1, E), full),
                  pl.BlockSpec((1, E), full)],
        out_specs=[pl.BlockSpec((TM, 2 * S), rowb),
                   pl.BlockSpec((TM, S), rowb),
                   pl.BlockSpec((TM, H), rowb),
                   pl.BlockSpec((TM, H), rowb),
                   pl.BlockSpec((TM, 1), rowb),
                   pl.BlockSpec((TM, 1), rowb),
                   pl.BlockSpec((TM, E), rowb)],
        compiler_params=pltpu.CompilerParams(dimension_semantics=("parallel",)),
        cost_estimate=pl.CostEstimate(
            flops=Np * 2 * (E * 3 * S + S * S + 4 * S * S + 3 * S * H + H * E),
            transcendentals=Np * (6 * S + H),
            bytes_accessed=4 * Np * (2 * E + 3 * S + 2 * H) + 4 * E * 3 * S
            + 2 * H * E + 4 * 3 * S * H),
    )(xp, w_cat.astype(jnp.bfloat16), b_cat, w_phase2, b_phase2, opc, wra, wrb, rel_b1,
      dec_w1, dec_b1, dec_ln1_g, dec_ln1_b, wd2_bf, dec_b2,
      dec_ln2_g, dec_ln2_b)

    TI = 32 if N % 32 == 0 else 8
    TJ = 1024 if N % 1024 == 0 else (512 if N % 512 == 0 else
                                     (256 if N % 256 == 0 else
                                      (128 if N % 128 == 0 else N)))
    Npr = _ceil_to(max(N, TI), TI)
    Npc = _ceil_to(max(N, TJ), TJ)
    Ap = ac[:N] if Npr == N else (
        jnp.zeros((Npr, H), jnp.float32).at[:N].set(ac[:N]))
    Bp = bc[:N] if Npc == N else (
        jnp.zeros((Npc, H), jnp.float32).at[:N].set(bc[:N]))
    vap = va[:N] if Npr == N else (
        jnp.zeros((Npr, 1), jnp.float32).at[:N].set(va[:N]))
    vbp = vb[:N] if Npc == N else (
        jnp.zeros((Npc, 1), jnp.float32).at[:N].set(vb[:N]))
    va3 = vap.reshape(Npr // TI, 1, TI)

    relt, sc, cf = pl.pallas_call(
        functools.partial(_rel_body, TI=TI, TJ=TJ, H=H, Q=Q),
        out_shape=(jax.ShapeDtypeStruct((Npr, Q, Npc), jnp.float32),
                   jax.ShapeDtypeStruct((Npr, Npc), jnp.float32),
                   jax.ShapeDtypeStruct((Npr, Npc), jnp.float32)),
        grid=(Npr // TI, Npc // TJ),
        in_specs=[pl.BlockSpec((TI, H), lambda i, j: (i, 0)),
                  pl.BlockSpec((TJ, H), lambda i, j: (j, 0)),
                  pl.BlockSpec((1, 1, TI), lambda i, j: (i, 0, 0)),
                  pl.BlockSpec((TJ, 1), lambda i, j: (j, 0)),
                  pl.BlockSpec((1, H), lambda i, j: (0, 0)),
                  pl.BlockSpec((1, H), lambda i, j: (0, 0)),
                  pl.BlockSpec((H, Q), lambda i, j: (0, 0)),
                  pl.BlockSpec((Q, 1), lambda i, j: (0, 0))],
        out_specs=[pl.BlockSpec((TI, Q, TJ), lambda i, j: (i, 0, j)),
                   pl.BlockSpec((TI, TJ), lambda i, j: (i, j)),
                   pl.BlockSpec((TI, TJ), lambda i, j: (i, j))],
        compiler_params=pltpu.CompilerParams(
            dimension_semantics=("parallel", "parallel")),
        cost_estimate=pl.CostEstimate(
            flops=Npr * Npc * (2 * H * Q + 14 * H + 4 * Q + 2 * H),
            transcendentals=Npr * Npc * H,
            bytes_accessed=4 * (Npr * H + Npc * H + Npr * Npc * (Q + 2))),
    )(Ap, Bp, va3, vbp, rel_ln_g, rel_ln_b, rel_w2,
      jnp.transpose(rel_b2))

    rel = jnp.transpose(relt, (0, 2, 1))
    if Npr != N or Npc != N:
        rel = rel[:N, :N]
        sc = sc[:N, :N]
        cf = cf[:N, :N]
    enhanced = dec if Np == N else dec[:N]
    meta = {"relation_states": rel,
            "relation_scores": sc,
            "confidence": cf,
            "quantum_states": {"real": st[:N, :S], "imag": st[:N, S:],
                               "phases": ph if Np == N else ph[:N]}}
    return enhanced, meta


# speedup vs baseline: 1.0772x; 1.0451x over previous
import functools
import math

import jax
import jax.numpy as jnp
from jax import lax
from jax.experimental import pallas as pl
from jax.experimental.pallas import tpu as pltpu

_GC1 = math.sqrt(2.0 / math.pi)
_GC2 = _GC1 * 0.044715


def _ln(x, g, b, eps=1e-5):
    mu = jnp.mean(x, axis=-1, keepdims=True)
    xc = x - mu
    v = jnp.mean(xc * xc, axis=-1, keepdims=True)
    return xc * lax.rsqrt(v + eps) * g + b


def _gelu(x):
    t = jnp.tanh(x * (_GC1 + _GC2 * (x * x)))
    return (0.5 * x) * (1.0 + t)


def _ceil_to(x, m):
    return (x + m - 1) // m * m


def _entity_body(x_ref, wcat_ref, bcat_ref, wp2_ref, bp2_ref, opc_ref,
                 wra_ref, wrb_ref, br1_ref, grel_ref,
                 wd1_ref, bd1_ref, g1_ref, e1_ref,
                 wd2_ref, bd2_ref, g2_ref, e2_ref,
                 st_ref, ph_ref, ac_ref, bc_ref, acg_ref, bcg_ref,
                 va_ref, vb_ref, dec_ref,
                 *, S):
    x = x_ref[...]
    t = jnp.tanh(jnp.dot(x.astype(jnp.bfloat16), wcat_ref[...],
                         preferred_element_type=jnp.float32) + bcat_ref[...])
    re = t[:, :S]
    im = t[:, S:2 * S]
    ph = math.pi * jnp.tanh(
        jnp.dot(t[:, 2 * S:], wp2_ref[...],
                preferred_element_type=jnp.float32) + bp2_ref[...])
    cp = jnp.cos(ph)
    sp = jnp.sin(ph)
    rw = re * cp - im * sp
    iw = re * sp + im * cp
    inv = lax.rsqrt(jnp.sum(rw * rw + iw * iw, axis=-1, keepdims=True) + 1e-12)
    st = jnp.dot(jnp.concatenate([rw * inv, iw * inv], axis=-1), opc_ref[...],
                 preferred_element_type=jnp.float32)
    st_ref[...] = st
    ph_ref[...] = ph

    real = st[:, :S]
    a = jnp.dot(real, wra_ref[...],
                preferred_element_type=jnp.float32) + br1_ref[...]
    b = jnp.dot(real, wrb_ref[...], preferred_element_type=jnp.float32)
    ac = a - jnp.mean(a, axis=-1, keepdims=True)
    bc = b - jnp.mean(b, axis=-1, keepdims=True)
    ac_ref[...] = ac
    bc_ref[...] = bc
    grel = grel_ref[...]
    acg_ref[...] = ac * grel
    bcg_ref[...] = bc * grel
    va_ref[...] = jnp.mean(ac * ac, axis=-1, keepdims=True)
    vb_ref[...] = jnp.mean(bc * bc, axis=-1, keepdims=True)

    hd = jnp.dot(real, wd1_ref[...],
                 preferred_element_type=jnp.float32) + bd1_ref[...]
    hd = _gelu(_ln(hd, g1_ref[...], e1_ref[...]))
    y = jnp.dot(hd.astype(jnp.bfloat16), wd2_ref[...],
                preferred_element_type=jnp.float32) + bd2_ref[...]
    dec_ref[...] = _ln(y, g2_ref[...], e2_ref[...])


def _rel_body(ac_ref, bc_ref, acg_ref, bcg_ref, va_ref, vb_ref, be_ref,
              w2_ref, b2t_ref,
              out_ref, sc_ref, cf_ref, *, TI, TJ, H, Q):
    bc = bc_ref[...]
    gram = lax.dot_general(bc, ac_ref[...],
                           (((1,), (1,)), ((), ())),
                           preferred_element_type=jnp.float32)
    rstd = lax.rsqrt(gram * (2.0 / H) + vb_ref[...] + va_ref[0] + 1e-5)

    bcg = bcg_ref[...]
    be = be_ref[...]
    w2 = w2_ref[...]
    b2t = b2t_ref[...]
    j0 = pl.program_id(1) * TJ
    jota = lax.broadcasted_iota(jnp.int32, (1, TJ), 1)
    for ii in range(TI):
        xc = acg_ref[ii:ii + 1, :] + bcg
        xn = xc * rstd[:, ii:ii + 1] + be
        u = _gelu(xn)
        ot = lax.dot_general(w2, u, (((0,), (1,)), ((), ())),
                             preferred_element_type=jnp.float32) + b2t
        out_ref[ii] = ot
        sc_ref[ii:ii + 1, :] = jnp.sum(ot, axis=0, keepdims=True)
        nrm = jnp.sqrt(jnp.sum(ot * ot, axis=0, keepdims=True)) * \
            (1.0 / math.sqrt(Q))
        cf = jnp.minimum(nrm, 1.0)
        gi = pl.program_id(0) * TI + ii
        cf_ref[ii:ii + 1, :] = jnp.where(jota == gi - j0, 0.0, cf)


def kernel(entity_emb, w_real, b_real, w_imag, b_imag, w_phase1, b_phase1,
           w_phase2, b_phase2, op_real, op_imag, op_blocks,
           rel_w1, rel_b1, rel_ln_g, rel_ln_b, rel_w2, rel_b2,
           dec_w1, dec_b1, dec_ln1_g, dec_ln1_b, dec_w2, dec_b2,
           dec_ln2_g, dec_ln2_b, normalization):
    x = jnp.asarray(entity_emb, jnp.float32)
    N, E = x.shape
    S = w_phase2.shape[0]
    H = rel_w1.shape[1]
    Q = rel_w2.shape[1]

    w_cat = jnp.concatenate([w_real, w_imag, w_phase1], axis=1)
    b_cat = jnp.concatenate([b_real, b_imag, b_phase1], axis=1)
    opc = op_blocks[0]
    for s in range(1, op_blocks.shape[0]):
        opc = jnp.dot(opc, op_blocks[s], preferred_element_type=jnp.float32)
    wra = rel_w1[:S, :]
    wrb = rel_w1[S:, :]
    wd2_bf = dec_w2.astype(jnp.bfloat16)

    TM = 128 if N % 128 == 0 else 8
    Np = _ceil_to(max(N, 8), TM)
    xp = x if Np == N else jnp.zeros((Np, E), jnp.float32).at[:N].set(x)

    full = lambda i: (0, 0)
    rowb = lambda i: (i, 0)
    st, ph, ac, bc, acg, bcg, va, vb, dec = pl.pallas_call(
        functools.partial(_entity_body, S=S),
        out_shape=(jax.ShapeDtypeStruct((Np, 2 * S), jnp.float32),
                   jax.ShapeDtypeStruct((Np, S), jnp.float32),
                   jax.ShapeDtypeStruct((Np, H), jnp.float32),
                   jax.ShapeDtypeStruct((Np, H), jnp.float32),
                   jax.ShapeDtypeStruct((Np, H), jnp.float32),
                   jax.ShapeDtypeStruct((Np, H), jnp.float32),
                   jax.ShapeDtypeStruct((Np, 1), jnp.float32),
                   jax.ShapeDtypeStruct((Np, 1), jnp.float32),
                   jax.ShapeDtypeStruct((Np, E), jnp.float32)),
        grid=(Np // TM,),
        in_specs=[pl.BlockSpec((TM, E), rowb),
                  pl.BlockSpec((E, 3 * S), full),
                  pl.BlockSpec((1, 3 * S), full),
                  pl.BlockSpec((S, S), full),
                  pl.BlockSpec((1, S), full),
                  pl.BlockSpec((2 * S, 2 * S), full),
                  pl.BlockSpec((S, H), full),
                  pl.BlockSpec((S, H), full),
                  pl.BlockSpec((1, H), full),
                  pl.BlockSpec((1, H), full),
                  pl.BlockSpec((S, H), full),
                  pl.BlockSpec((1, H), full),
                  pl.BlockSpec((1, H), full),
                  pl.BlockSpec((1, H), full),
                  pl.BlockSpec((H, E), full),
                  pl.BlockSpec((1, E), full),
                  pl.BlockSpec((1, E), full),
                  pl.BlockSpec((1, E), full)],
        out_specs=[pl.BlockSpec((TM, 2 * S), rowb),
                   pl.BlockSpec((TM, S), rowb),
                   pl.BlockSpec((TM, H), rowb),
                   pl.BlockSpec((TM, H), rowb),
                   pl.BlockSpec((TM, H), rowb),
                   pl.BlockSpec((TM, H), rowb),
                   pl.BlockSpec((TM, 1), rowb),
                   pl.BlockSpec((TM, 1), rowb),
                   pl.BlockSpec((TM, E), rowb)],
        compiler_params=pltpu.CompilerParams(dimension_semantics=("parallel",)),
        cost_estimate=pl.CostEstimate(
            flops=Np * 2 * (E * 3 * S + S * S + 4 * S * S + 3 * S * H + H * E),
            transcendentals=Np * (6 * S + H),
            bytes_accessed=4 * Np * (2 * E + 3 * S + 2 * H) + 4 * E * 3 * S
            + 2 * H * E + 4 * 3 * S * H),
    )(xp, w_cat.astype(jnp.bfloat16), b_cat, w_phase2, b_phase2, opc, wra, wrb, rel_b1,
      rel_ln_g, dec_w1, dec_b1, dec_ln1_g, dec_ln1_b, wd2_bf, dec_b2,
      dec_ln2_g, dec_ln2_b)

    TI = 32 if N % 32 == 0 else 8
    TJ = 1024 if N % 1024 == 0 else (512 if N % 512 == 0 else
                                     (256 if N % 256 == 0 else
                                      (128 if N % 128 == 0 else N)))
    Npr = _ceil_to(max(N, TI), TI)
    Npc = _ceil_to(max(N, TJ), TJ)
    Ap = ac[:N] if Npr == N else (
        jnp.zeros((Npr, H), jnp.float32).at[:N].set(ac[:N]))
    Bp = bc[:N] if Npc == N else (
        jnp.zeros((Npc, H), jnp.float32).at[:N].set(bc[:N]))
    vap = va[:N] if Npr == N else (
        jnp.zeros((Npr, 1), jnp.float32).at[:N].set(va[:N]))
    vbp = vb[:N] if Npc == N else (
        jnp.zeros((Npc, 1), jnp.float32).at[:N].set(vb[:N]))
    Agp = acg[:N] if Npr == N else (
        jnp.zeros((Npr, H), jnp.float32).at[:N].set(acg[:N]))
    Bgp = bcg[:N] if Npc == N else (
        jnp.zeros((Npc, H), jnp.float32).at[:N].set(bcg[:N]))
    va3 = vap.reshape(Npr // TI, 1, TI)

    relt, sc, cf = pl.pallas_call(
        functools.partial(_rel_body, TI=TI, TJ=TJ, H=H, Q=Q),
        out_shape=(jax.ShapeDtypeStruct((Npr, Q, Npc), jnp.float32),
                   jax.ShapeDtypeStruct((Npr, Npc), jnp.float32),
                   jax.ShapeDtypeStruct((Npr, Npc), jnp.float32)),
        grid=(Npr // TI, Npc // TJ),
        in_specs=[pl.BlockSpec((TI, H), lambda i, j: (i, 0)),
                  pl.BlockSpec((TJ, H), lambda i, j: (j, 0)),
                  pl.BlockSpec((TI, H), lambda i, j: (i, 0)),
                  pl.BlockSpec((TJ, H), lambda i, j: (j, 0)),
                  pl.BlockSpec((1, 1, TI), lambda i, j: (i, 0, 0)),
                  pl.BlockSpec((TJ, 1), lambda i, j: (j, 0)),
                  pl.BlockSpec((1, H), lambda i, j: (0, 0)),
                  pl.BlockSpec((H, Q), lambda i, j: (0, 0)),
                  pl.BlockSpec((Q, 1), lambda i, j: (0, 0))],
        out_specs=[pl.BlockSpec((TI, Q, TJ), lambda i, j: (i, 0, j)),
                   pl.BlockSpec((TI, TJ), lambda i, j: (i, j)),
                   pl.BlockSpec((TI, TJ), lambda i, j: (i, j))],
        compiler_params=pltpu.CompilerParams(
            dimension_semantics=("parallel", "parallel")),
        cost_estimate=pl.CostEstimate(
            flops=Npr * Npc * (2 * H * Q + 14 * H + 4 * Q + 2 * H),
            transcendentals=Npr * Npc * H,
            bytes_accessed=4 * (Npr * H + Npc * H + Npr * Npc * (Q + 2))),
    )(Ap, Bp, Agp, Bgp, va3, vbp, rel_ln_b, rel_w2,
      jnp.transpose(rel_b2))

    rel = jnp.transpose(relt, (0, 2, 1))
    if Npr != N or Npc != N:
        rel = rel[:N, :N]
        sc = sc[:N, :N]
        cf = cf[:N, :N]
    enhanced = dec if Np == N else dec[:N]
    meta = {"relation_states": rel,
            "relation_scores": sc,
            "confidence": cf,
            "quantum_states": {"real": st[:N, :S], "imag": st[:N, S:],
                               "phases": ph if Np == N else ph[:N]}}
    return enhanced, meta
